# baseline (speedup 1.0000x reference)
import functools

import jax
import jax.numpy as jnp
from jax import lax
from jax.experimental import pallas as pl
from jax.experimental.pallas import tpu as pltpu
from jax.experimental.pallas import tpu_sc as plsc

N = 50000
E = 800000
H = 64
NLAYERS = 4

NC, NS = 2, 16
WT = NC * NS

NR = 49
RR = 1024
CAP = 768
TSPAN = NR * CAP
DUMPBASE = WT * TSPAN
SLOTS = DUMPBASE + 4096
AGG_ROWS = NR * RR

EPW = E // WT
SB = 128
NSB = EPW // SB
LB = 256

_SC_PARAMS = pltpu.CompilerParams(use_tc_tiling_on_sc=False)


def _part_body(src_hbm, dst_hbm, ea0_hbm, ea1_hbm, ea2_hbm, ea3_hbm,
               pk_hbm, p0_hbm, p1_hbm, p2_hbm, p3_hbm,
               fbuf, zbuf, cnt2, srcv, dstv, e0v, e1v, e2v, e3v,
               pbuf, posb, sem, sem2):
    ci = lax.axis_index("c")
    si = lax.axis_index("s")
    t = si * NC + ci
    lane = lax.iota(jnp.int32, 16)
    benign = lane * 0 + (RR << 17)
    zero16 = (lane * 0).astype(jnp.float32)

    def frow(i, c2):
        fbuf[pl.ds(i * 16, 16)] = benign
        zbuf[pl.ds(i * 16, 16)] = zero16
        return c2

    lax.fori_loop(0, 256, frow, 0)

    def zcnt(i, c2):
        for j in range(4):
            cnt2[i, pl.ds(j * 16, 16)] = lane * 0
        return c2

    lax.fori_loop(0, NR, zcnt, 0)

    tb = t * TSPAN

    def fill(kk, c2):
        off = pl.multiple_of(tb + kk * 3136, 8)
        pltpu.sync_copy(fbuf.at[pl.ds(0, 3136)], pk_hbm.at[pl.ds(off, 3136)])
        pltpu.sync_copy(zbuf.at[pl.ds(0, 3136)], p0_hbm.at[pl.ds(off, 3136)])
        pltpu.sync_copy(zbuf.at[pl.ds(0, 3136)], p1_hbm.at[pl.ds(off, 3136)])
        pltpu.sync_copy(zbuf.at[pl.ds(0, 3136)], p2_hbm.at[pl.ds(off, 3136)])
        pltpu.sync_copy(zbuf.at[pl.ds(0, 3136)], p3_hbm.at[pl.ds(off, 3136)])
        return c2

    lax.fori_loop(0, TSPAN // 3136, fill, 0)
    db = pl.multiple_of(DUMPBASE + t * SB, 8)
    pltpu.sync_copy(fbuf.at[pl.ds(0, SB)], pk_hbm.at[pl.ds(db, SB)])
    pltpu.sync_copy(zbuf.at[pl.ds(0, SB)], p0_hbm.at[pl.ds(db, SB)])
    pltpu.sync_copy(zbuf.at[pl.ds(0, SB)], p1_hbm.at[pl.ds(db, SB)])
    pltpu.sync_copy(zbuf.at[pl.ds(0, SB)], p2_hbm.at[pl.ds(db, SB)])
    pltpu.sync_copy(zbuf.at[pl.ds(0, SB)], p3_hbm.at[pl.ds(db, SB)])

    def batch(g, carry):
        last = g >= NSB
        e0 = pl.multiple_of(t * EPW + jnp.where(last, EPW - SB, g * SB), 8)
        vfrom = jnp.where(last, SB - (EPW - NSB * SB), 0)
        c1 = pltpu.async_copy(src_hbm.at[pl.ds(e0, SB)], srcv, sem2)
        c2_ = pltpu.async_copy(dst_hbm.at[pl.ds(e0, SB)], dstv, sem2)
        c3 = pltpu.async_copy(ea0_hbm.at[pl.ds(e0, SB)], e0v, sem2)
        c4 = pltpu.async_copy(ea1_hbm.at[pl.ds(e0, SB)], e1v, sem2)
        c5 = pltpu.async_copy(ea2_hbm.at[pl.ds(e0, SB)], e2v, sem2)
        c6 = pltpu.async_copy(ea3_hbm.at[pl.ds(e0, SB)], e3v, sem2)
        c1.wait(); c2_.wait(); c3.wait(); c4.wait(); c5.wait(); c6.wait()
        for gi in range(SB // 16):
            sl = pl.ds(gi * 16, 16)
            d = dstv[sl]
            s = srcv[sl]
            r = d >> 10
            dl = d - r * RR
            pbuf[sl] = s | (dl << 17)
            posv = lane * 0
            for k in range(16):
                rk = r[k]
                crow = cnt2[rk, pl.ds(0, 16)]
                c0 = crow[0]
                okk = (c0 < CAP) & (gi * 16 + k >= vfrom)
                pos_k = jnp.where(okk, t * TSPAN + rk * CAP + c0,
                                  DUMPBASE + t * SB + k)
                cnt2[rk, pl.ds(0, 16)] = crow + jnp.where(okk, 1, 0)
                posv = jnp.where(lane == k, pos_k, posv)
            posb[sl] = posv
        s1 = pltpu.async_copy(pbuf, pk_hbm.at[posb], sem)
        s2 = pltpu.async_copy(e0v, p0_hbm.at[posb], sem)
        s3 = pltpu.async_copy(e1v, p1_hbm.at[posb], sem)
        s4 = pltpu.async_copy(e2v, p2_hbm.at[posb], sem)
        s5 = pltpu.async_copy(e3v, p3_hbm.at[posb], sem)
        s1.wait(); s2.wait(); s3.wait(); s4.wait(); s5.wait()
        return carry

    lax.fori_loop(0, NSB + 1, batch, 0)


def _partition(src, dst, ea0, ea1, ea2, ea3):
    mesh = plsc.VectorSubcoreMesh(core_axis_name="c", subcore_axis_name="s")
    i32 = jnp.int32
    f32 = jnp.float32
    fn = pl.kernel(
        _part_body,
        mesh=mesh,
        out_type=[jax.ShapeDtypeStruct((SLOTS,), i32)]
                 + [jax.ShapeDtypeStruct((SLOTS,), f32)] * 4,
        scratch_types=[
            pltpu.VMEM((4096,), i32),
            pltpu.VMEM((4096,), f32),
            pltpu.VMEM((NR, 64), i32),
            pltpu.VMEM((SB,), i32),
            pltpu.VMEM((SB,), i32),
            pltpu.VMEM((SB,), f32),
            pltpu.VMEM((SB,), f32),
            pltpu.VMEM((SB,), f32),
            pltpu.VMEM((SB,), f32),
            pltpu.VMEM((SB,), i32),
            pltpu.VMEM((SB,), i32),
            pltpu.SemaphoreType.DMA,
            pltpu.SemaphoreType.DMA,
        ],
        compiler_params=_SC_PARAMS,
    )
    return fn(src, dst, ea0, ea1, ea2, ea3)


def _layer_body(pk_hbm, c_hbm, y_hbm, agg_hbm,
                acc, rows, cbuf, pkv, dlb, gia, gib, sem):
    ci = lax.axis_index("c")
    si = lax.axis_index("s")
    w = si * NC + ci
    lane = lax.iota(jnp.int32, 16)
    zero16 = (lane * 0).astype(jnp.float32)

    for rsub in range(2):
        r = w + rsub * WT

        def zrow(i, c2):
            for j in range(4):
                acc[i, pl.ds(j * 16, 16)] = zero16
            return c2

        lax.fori_loop(0, RR + 8, zrow, 0)

        def cell(tcell, carry):
            cb = tcell * TSPAN + r * CAP
            for sub in range(CAP // LB):
                e0 = pl.multiple_of(cb + sub * LB, 8)
                pltpu.sync_copy(pk_hbm.at[pl.ds(e0, LB)], pkv)
                pltpu.sync_copy(c_hbm.at[pl.ds(e0, LB)], cbuf)
                for gi in range(LB // 16):
                    sl = pl.ds(gi * 16, 16)
                    pk = pkv[sl]
                    if gi < 8:
                        gia[pl.ds(gi * 16, 16)] = pk & 0x1FFFF
                    else:
                        gib[pl.ds(gi * 16 - 128, 16)] = pk & 0x1FFFF
                    dlb[sl] = pk >> 17
                g1 = pltpu.async_copy(y_hbm.at[gia],
                                      rows.at[pl.ds(0, 128)], sem)
                g2 = pltpu.async_copy(y_hbm.at[gib],
                                      rows.at[pl.ds(128, 128)], sem)
                g1.wait()
                g2.wait()

                def mrow(i, c2):
                    for j in range(4):
                        sl2 = pl.ds(j * 16, 16)
                        cbuf[i, sl2] = jnp.maximum(
                            rows[i, sl2] + cbuf[i, sl2], 0.0)
                    return c2

                lax.fori_loop(0, LB, mrow, 0)

                def accg(gg, c2):
                    dlv = dlb[pl.ds(pl.multiple_of(gg * 16, 16), 16)]
                    for k in range(16):
                        dl = dlv[k]
                        i = gg * 16 + k
                        for j in range(4):
                            sl2 = pl.ds(j * 16, 16)
                            acc[dl, sl2] = acc[dl, sl2] + cbuf[i, sl2]
                    return c2

                lax.fori_loop(0, LB // 16, accg, 0)
            return carry

        @pl.when((r < NR) | (rsub == 0))
        def _do_range():
            lax.fori_loop(0, WT, cell, 0)
            pltpu.sync_copy(acc.at[pl.ds(0, RR)],
                            agg_hbm.at[pl.ds(r * RR, RR)])


def _sc_layer(pk, c, y):
    mesh = plsc.VectorSubcoreMesh(core_axis_name="c", subcore_axis_name="s")
    fn = pl.kernel(
        _layer_body,
        mesh=mesh,
        out_type=jax.ShapeDtypeStruct((AGG_ROWS, H), jnp.float32),
        scratch_types=[
            pltpu.VMEM((RR + 8, H), jnp.float32),
            pltpu.VMEM((LB, H), jnp.float32),
            pltpu.VMEM((LB, H), jnp.float32),
            pltpu.VMEM((LB,), jnp.int32),
            pltpu.VMEM((LB,), jnp.int32),
            pltpu.VMEM((128,), jnp.int32),
            pltpu.VMEM((128,), jnp.int32),
            pltpu.SemaphoreType.DMA,
        ],
        compiler_params=_SC_PARAMS,
    )
    return fn(pk, c, y)


_RB = 1000
_EB = 4096


def _proj_body(h_ref, win_ref, bin_ref, wy_ref, by_ref, wz_ref, bz_ref,
               x_ref, y_ref, z_ref):
    x = jnp.dot(h_ref[...], win_ref[...],
                preferred_element_type=jnp.float32) + bin_ref[...]
    x_ref[...] = x
    y_ref[...] = jnp.dot(x, wy_ref[...],
                         preferred_element_type=jnp.float32) + by_ref[...]
    z_ref[...] = jnp.dot(x, wz_ref[...],
                         preferred_element_type=jnp.float32) + bz_ref[...]


def _proj(h, win, bin_, wy, by, wz, bz):
    out = jax.ShapeDtypeStruct((N, H), jnp.float32)
    w2 = pl.BlockSpec((H, H), lambda i: (0, 0))
    bspec = pl.BlockSpec((1, H), lambda i: (0, 0))
    return pl.pallas_call(
        _proj_body,
        grid=(N // _RB,),
        in_specs=[pl.BlockSpec((_RB, 128), lambda i: (i, 0)),
                  pl.BlockSpec((128, H), lambda i: (0, 0)),
                  bspec, w2, bspec, w2, bspec],
        out_specs=[pl.BlockSpec((_RB, H), lambda i: (i, 0))] * 3,
        out_shape=[out, out, out],
    )(h, win, bin_, wy, by, wz, bz)


def _cterm_body(ea_ref, wme_ref, c_ref):
    c_ref[...] = jnp.dot(ea_ref[...], wme_ref[...],
                         preferred_element_type=jnp.float32)


def _cterm(ea, wme):
    return pl.pallas_call(
        _cterm_body,
        grid=(SLOTS // _EB,),
        in_specs=[pl.BlockSpec((_EB, 4), lambda i: (i, 0)),
                  pl.BlockSpec((4, H), lambda i: (0, 0))],
        out_specs=pl.BlockSpec((_EB, H), lambda i: (i, 0)),
        out_shape=jax.ShapeDtypeStruct((SLOTS, H), jnp.float32),
    )(ea, wme)


def _combine_body(x_ref, z_ref, agg_ref, wy_ref, by_ref, wz_ref, bz_ref,
                  xn_ref, y_ref, zn_ref):
    xn = x_ref[...] + jnp.maximum(z_ref[...] + agg_ref[...], 0.0)
    xn_ref[...] = xn
    y_ref[...] = jnp.dot(xn, wy_ref[...],
                         preferred_element_type=jnp.float32) + by_ref[...]
    zn_ref[...] = jnp.dot(xn, wz_ref[...],
                          preferred_element_type=jnp.float32) + bz_ref[...]


def _combine(x, z, agg, wy, by, wz, bz):
    row = pl.BlockSpec((_RB, H), lambda i: (i, 0))
    wspec = pl.BlockSpec((H, H), lambda i: (0, 0))
    bspec = pl.BlockSpec((1, H), lambda i: (0, 0))
    out = jax.ShapeDtypeStruct((N, H), jnp.float32)
    return pl.pallas_call(
        _combine_body,
        grid=(N // _RB,),
        in_specs=[row, row, row, wspec, bspec, wspec, bspec],
        out_specs=[row, row, row],
        out_shape=[out, out, out],
    )(x, z, agg, wy, by, wz, bz)


def _final_body(x_ref, z_ref, agg_ref, wo_ref, bo_ref, out_ref):
    xn = x_ref[...] + jnp.maximum(z_ref[...] + agg_ref[...], 0.0)
    out_ref[...] = jnp.dot(xn, wo_ref[...],
                           preferred_element_type=jnp.float32) + bo_ref[...]


def _final(x, z, agg, wo_pad, bo_pad):
    row = pl.BlockSpec((_RB, H), lambda i: (i, 0))
    return pl.pallas_call(
        _final_body,
        grid=(N // _RB,),
        in_specs=[row, row, row,
                  pl.BlockSpec((H, 128), lambda i: (0, 0)),
                  pl.BlockSpec((1, 128), lambda i: (0, 0))],
        out_specs=pl.BlockSpec((_RB, 128), lambda i: (i, 0)),
        out_shape=jax.ShapeDtypeStruct((N, 128), jnp.float32),
    )(x, z, agg, wo_pad, bo_pad)


def kernel(h, edge_index, edge_attr, Win, bin_, Wmsg, bmsg, Wself, bself,
           Wout, bout):
    src = edge_index[0]
    dst = edge_index[1]
    b2 = lambda v: v.reshape(1, -1)

    wy = [Wmsg[i, :H, :] for i in range(NLAYERS)]
    wme = [Wmsg[i, H:, :] for i in range(NLAYERS)]
    wz = [Wself[i] for i in range(NLAYERS)]

    pk, p0, p1, p2, p3 = _partition(src, dst,
                                    edge_attr[:, 0], edge_attr[:, 1],
                                    edge_attr[:, 2], edge_attr[:, 3])
    ea_perm = jnp.stack([p0, p1, p2, p3], axis=1)

    x, y, z = _proj(h, Win, b2(bin_), wy[0], b2(bmsg[0]), wz[0], b2(bself[0]))
    for i in range(NLAYERS):
        c = _cterm(ea_perm, wme[i])
        agg = _sc_layer(pk, c, y)[:N]
        if i + 1 < NLAYERS:
            x, y, z = _combine(x, z, agg, wy[i + 1], b2(bmsg[i + 1]),
                               wz[i + 1], b2(bself[i + 1]))
        else:
            wo_pad = jnp.zeros((H, 128), jnp.float32).at[:, :3].set(Wout)
            bo_pad = jnp.zeros((1, 128), jnp.float32).at[:, :3].set(bout)
            out_pad = _final(x, z, agg, wo_pad, bo_pad)
    return out_pad[:, :3]
